# Initial kernel scaffold; baseline (speedup 1.0000x reference)
import jax
import jax.numpy as jnp
from jax.experimental import pallas as pl


def kernel(x, emb0, emb1, emb2, emb3, emb4, emb5, emb6, emb7, emb8, W, b, gamma, beta):
    raise NotImplementedError("write your pallas kernel here")


# speedup vs baseline: 10.4238x; 10.4238x over previous
import jax
import jax.numpy as jnp
from jax.experimental import pallas as pl

_EMBED = [(33, 17), (2, 1), (8, 4), (3, 2), (3, 2), (3, 2), (3, 2), (3, 2), (20, 10)]
_VOFF = [0, 33, 35, 43, 46, 49, 52, 55, 58]
_DOFF = [0, 17, 18, 22, 24, 26, 28, 30, 32]
_B = 16384
_BLK = 2048
_CONT_COL = 78


def _body(x_ref, cont_ref, tpad_ref, wpad_ref, bias_ref, gamma_ref, beta_ref,
          out_ref):
    c = cont_ref[...]
    mean = jnp.mean(c)
    var = jnp.mean(c * c) - mean * mean
    inv = jax.lax.rsqrt(var + 1e-5)

    xb = x_ref[...]
    cn = (xb[:, 0:1] - mean) * inv * gamma_ref[...] + beta_ref[...]

    col = jax.lax.broadcasted_iota(jnp.int32, (_BLK, 128), 1)
    f = jnp.where(col == _CONT_COL, cn, 0.0)
    for i in range(9):
        tgt = xb[:, i + 1:i + 2].astype(jnp.int32) + _VOFF[i]
        f = f + (col == tgt).astype(jnp.float32)

    h = jnp.dot(f, tpad_ref[...], preferred_element_type=jnp.float32)
    z = jnp.dot(h, wpad_ref[...], preferred_element_type=jnp.float32)
    z = z + bias_ref[...]
    m = jnp.max(z, axis=1, keepdims=True)
    e = jnp.exp(z - m)
    out_ref[...] = e / jnp.sum(e, axis=1, keepdims=True)


def kernel(x, emb0, emb1, emb2, emb3, emb4, emb5, emb6, emb7, emb8, W, b,
           gamma, beta):
    tables = [emb0, emb1, emb2, emb3, emb4, emb5, emb6, emb7, emb8]
    tpad = jnp.zeros((128, 128), jnp.float32)
    for i, (v, d) in enumerate(_EMBED):
        tpad = tpad.at[_VOFF[i]:_VOFF[i] + v, _DOFF[i]:_DOFF[i] + d].set(tables[i])
    tpad = tpad.at[_CONT_COL, 42].set(1.0)
    wpad = jnp.zeros((128, 256), jnp.float32).at[:43, :].set(W.T)

    cont_full = x[:, 0].reshape(8, 2048)
    grid = _B // _BLK

    out = pl.pallas_call(
        _body,
        grid=(grid,),
        in_specs=[
            pl.BlockSpec((_BLK, 10), lambda j: (j, 0)),
            pl.BlockSpec((8, 2048), lambda j: (0, 0)),
            pl.BlockSpec((128, 128), lambda j: (0, 0)),
            pl.BlockSpec((128, 256), lambda j: (0, 0)),
            pl.BlockSpec((1, 256), lambda j: (0, 0)),
            pl.BlockSpec((1, 1), lambda j: (0, 0)),
            pl.BlockSpec((1, 1), lambda j: (0, 0)),
        ],
        out_specs=pl.BlockSpec((_BLK, 256), lambda j: (j, 0)),
        out_shape=jax.ShapeDtypeStruct((_B, 256), jnp.float32),
    )(x, cont_full, tpad, wpad, b.reshape(1, 256), gamma.reshape(1, 1),
      beta.reshape(1, 1))
    return out


# speedup vs baseline: 10.6947x; 1.0260x over previous
import jax
import jax.numpy as jnp
from jax.experimental import pallas as pl

_EMBED = [(33, 17), (2, 1), (8, 4), (3, 2), (3, 2), (3, 2), (3, 2), (3, 2), (20, 10)]
_VOFF = [0, 33, 35, 43, 46, 49, 52, 55, 58]
_DOFF = [0, 17, 18, 22, 24, 26, 28, 30, 32]
_B = 16384
_BLK = 2048
_CONT_COL = 78


def _body(x_ref, cont_ref, tpad_ref, wpad_ref, bias_ref, gamma_ref, beta_ref,
          out_ref):
    c = cont_ref[...]
    mean = jnp.mean(c)
    var = jnp.mean(c * c) - mean * mean
    inv = jax.lax.rsqrt(var + 1e-5)

    xb = x_ref[...]
    cn = (xb[:, 0:1] - mean) * inv * gamma_ref[...] + beta_ref[...]

    col = jax.lax.broadcasted_iota(jnp.int32, (_BLK, 128), 1)
    f = jnp.where(col == _CONT_COL, cn, 0.0)
    for i in range(9):
        tgt = xb[:, i + 1:i + 2].astype(jnp.int32) + _VOFF[i]
        f = f + (col == tgt).astype(jnp.float32)

    m = jnp.dot(tpad_ref[...], wpad_ref[...],
                preferred_element_type=jnp.float32)
    z = jnp.dot(f.astype(jnp.bfloat16), m.astype(jnp.bfloat16),
                preferred_element_type=jnp.float32)
    z = z + bias_ref[...]
    m = jnp.max(z, axis=1, keepdims=True)
    e = jnp.exp(z - m)
    out_ref[...] = e / jnp.sum(e, axis=1, keepdims=True)


def kernel(x, emb0, emb1, emb2, emb3, emb4, emb5, emb6, emb7, emb8, W, b,
           gamma, beta):
    tables = [emb0, emb1, emb2, emb3, emb4, emb5, emb6, emb7, emb8]
    tpad = jnp.zeros((128, 128), jnp.float32)
    for i, (v, d) in enumerate(_EMBED):
        tpad = tpad.at[_VOFF[i]:_VOFF[i] + v, _DOFF[i]:_DOFF[i] + d].set(tables[i])
    tpad = tpad.at[_CONT_COL, 42].set(1.0)
    wpad = jnp.zeros((128, 256), jnp.float32).at[:43, :].set(W.T)

    cont_full = x[:, 0].reshape(8, 2048)
    grid = _B // _BLK

    out = pl.pallas_call(
        _body,
        grid=(grid,),
        in_specs=[
            pl.BlockSpec((_BLK, 10), lambda j: (j, 0)),
            pl.BlockSpec((8, 2048), lambda j: (0, 0)),
            pl.BlockSpec((128, 128), lambda j: (0, 0)),
            pl.BlockSpec((128, 256), lambda j: (0, 0)),
            pl.BlockSpec((1, 256), lambda j: (0, 0)),
            pl.BlockSpec((1, 1), lambda j: (0, 0)),
            pl.BlockSpec((1, 1), lambda j: (0, 0)),
        ],
        out_specs=pl.BlockSpec((_BLK, 256), lambda j: (j, 0)),
        out_shape=jax.ShapeDtypeStruct((_B, 256), jnp.float32),
    )(x, cont_full, tpad, wpad, b.reshape(1, 256), gamma.reshape(1, 1),
      beta.reshape(1, 1))
    return out


# speedup vs baseline: 18.0740x; 1.6900x over previous
import jax
import jax.numpy as jnp
from jax.experimental import pallas as pl

_EMBED = [(33, 17), (2, 1), (8, 4), (3, 2), (3, 2), (3, 2), (3, 2), (3, 2), (20, 10)]
_VOFF = [0, 33, 35, 43, 46, 49, 52, 55, 58]
_DOFF = [0, 17, 18, 22, 24, 26, 28, 30, 32]
_B = 16384
_BLK = 2048
_CONT_COL = 78


def _body(xt_ref, cont_ref, tpad_ref, wpad_ref, bias_ref, gamma_ref, beta_ref,
          out_ref):
    c = cont_ref[...]
    mean = jnp.mean(c)
    var = jnp.mean(c * c) - mean * mean
    inv = jax.lax.rsqrt(var + 1e-5)

    xt = xt_ref[...]
    cn = (xt[0:1, :] - mean) * inv * gamma_ref[...] + beta_ref[...]

    row = jax.lax.broadcasted_iota(jnp.int32, (128, _BLK), 0)
    ft = jnp.where(row == _CONT_COL, cn, 0.0)
    for i in range(9):
        tgt = xt[i + 1:i + 2, :].astype(jnp.int32) + _VOFF[i]
        ft = ft + jnp.where(row == tgt, 1.0, 0.0)
    ft = ft.astype(jnp.bfloat16)

    m = jnp.dot(tpad_ref[...], wpad_ref[...],
                preferred_element_type=jnp.float32)
    z = jax.lax.dot_general(ft, m.astype(jnp.bfloat16),
                            dimension_numbers=(((0,), (0,)), ((), ())),
                            preferred_element_type=jnp.float32)
    z = z + bias_ref[...]
    mx = jnp.max(z, axis=1, keepdims=True)
    e = jnp.exp(z - mx)
    out_ref[...] = e / jnp.sum(e, axis=1, keepdims=True)


def kernel(x, emb0, emb1, emb2, emb3, emb4, emb5, emb6, emb7, emb8, W, b,
           gamma, beta):
    tables = [emb0, emb1, emb2, emb3, emb4, emb5, emb6, emb7, emb8]
    tpad = jnp.zeros((128, 128), jnp.float32)
    for i, (v, d) in enumerate(_EMBED):
        tpad = tpad.at[_VOFF[i]:_VOFF[i] + v, _DOFF[i]:_DOFF[i] + d].set(tables[i])
    tpad = tpad.at[_CONT_COL, 42].set(1.0)
    wpad = jnp.zeros((128, 256), jnp.float32).at[:43, :].set(W.T)

    xt = x.T
    cont_full = x[:, 0].reshape(8, 2048)
    grid = _B // _BLK

    out = pl.pallas_call(
        _body,
        grid=(grid,),
        in_specs=[
            pl.BlockSpec((10, _BLK), lambda j: (0, j)),
            pl.BlockSpec((8, 2048), lambda j: (0, 0)),
            pl.BlockSpec((128, 128), lambda j: (0, 0)),
            pl.BlockSpec((128, 256), lambda j: (0, 0)),
            pl.BlockSpec((1, 256), lambda j: (0, 0)),
            pl.BlockSpec((1, 1), lambda j: (0, 0)),
            pl.BlockSpec((1, 1), lambda j: (0, 0)),
        ],
        out_specs=pl.BlockSpec((_BLK, 256), lambda j: (j, 0)),
        out_shape=jax.ShapeDtypeStruct((_B, 256), jnp.float32),
    )(xt, cont_full, tpad, wpad, b.reshape(1, 256), gamma.reshape(1, 1),
      beta.reshape(1, 1))
    return out


# speedup vs baseline: 19.1011x; 1.0568x over previous
import jax
import jax.numpy as jnp
from jax.experimental import pallas as pl

_EMBED = [(33, 17), (2, 1), (8, 4), (3, 2), (3, 2), (3, 2), (3, 2), (3, 2), (20, 10)]
_VOFF = [0, 33, 35, 43, 46, 49, 52, 55, 58]
_DOFF = [0, 17, 18, 22, 24, 26, 28, 30, 32]
_B = 16384
_BLK = 2048
_NV = 80


def _body(xt_ref, cont_ref, s_ref, tpad_ref, wpad_ref, wc_ref, bias_ref,
          gamma_ref, beta_ref, out_ref):
    c = cont_ref[...]
    mean = jnp.mean(c)
    var = jnp.mean(c * c) - mean * mean
    inv = jax.lax.rsqrt(var + 1e-5)

    xt = xt_ref[...]
    cn = (xt[0:1, :] - mean) * inv * gamma_ref[...] + beta_ref[...]

    xa = jnp.concatenate([xt[1:10, :], jnp.ones((1, _BLK), jnp.float32)], 0)
    tgt = jnp.dot(s_ref[...], xa, preferred_element_type=jnp.float32)
    rowf = jax.lax.broadcasted_iota(jnp.int32, (_NV, _BLK), 0).astype(jnp.float32)
    ft = jnp.where(tgt == rowf, 1.0, 0.0).astype(jnp.bfloat16)

    m = jnp.dot(tpad_ref[...], wpad_ref[...],
                preferred_element_type=jnp.float32)
    z = jax.lax.dot_general(ft, m.astype(jnp.bfloat16),
                            dimension_numbers=(((0,), (0,)), ((), ())),
                            preferred_element_type=jnp.float32)
    z = z + jax.lax.dot_general(cn, wc_ref[...],
                                dimension_numbers=(((0,), (0,)), ((), ())),
                                preferred_element_type=jnp.float32)
    z = z + bias_ref[...]
    mx = jnp.max(z, axis=1, keepdims=True)
    e = jnp.exp(z - mx)
    out_ref[...] = e / jnp.sum(e, axis=1, keepdims=True)


def kernel(x, emb0, emb1, emb2, emb3, emb4, emb5, emb6, emb7, emb8, W, b,
           gamma, beta):
    tables = [emb0, emb1, emb2, emb3, emb4, emb5, emb6, emb7, emb8]
    tpad = jnp.zeros((_NV, 128), jnp.float32)
    for i, (v, d) in enumerate(_EMBED):
        tpad = tpad.at[_VOFF[i]:_VOFF[i] + v, _DOFF[i]:_DOFF[i] + d].set(tables[i])
    wpad = jnp.zeros((128, 256), jnp.float32).at[:42, :].set(W[:, :42].T)

    import numpy as np
    s_np = np.zeros((_NV, 10), np.float32)
    for i, (v, _) in enumerate(_EMBED):
        s_np[_VOFF[i]:_VOFF[i] + v, i] = 1.0
        s_np[_VOFF[i]:_VOFF[i] + v, 9] = _VOFF[i]
    s_np[78:, 9] = -1.0
    s = jnp.asarray(s_np)

    xt = x.T
    cont_full = x[:, 0].reshape(8, 2048)
    grid = _B // _BLK

    out = pl.pallas_call(
        _body,
        grid=(grid,),
        in_specs=[
            pl.BlockSpec((10, _BLK), lambda j: (0, j)),
            pl.BlockSpec((8, 2048), lambda j: (0, 0)),
            pl.BlockSpec((_NV, 10), lambda j: (0, 0)),
            pl.BlockSpec((_NV, 128), lambda j: (0, 0)),
            pl.BlockSpec((128, 256), lambda j: (0, 0)),
            pl.BlockSpec((1, 256), lambda j: (0, 0)),
            pl.BlockSpec((1, 256), lambda j: (0, 0)),
            pl.BlockSpec((1, 1), lambda j: (0, 0)),
            pl.BlockSpec((1, 1), lambda j: (0, 0)),
        ],
        out_specs=pl.BlockSpec((_BLK, 256), lambda j: (j, 0)),
        out_shape=jax.ShapeDtypeStruct((_B, 256), jnp.float32),
    )(xt, cont_full, s, tpad, wpad, W[:, 42].reshape(1, 256),
      b.reshape(1, 256), gamma.reshape(1, 1), beta.reshape(1, 1))
    return out
